# baseline (speedup 1.0000x reference)
import jax
import jax.numpy as jnp
from jax import lax
from jax.experimental import pallas as pl

_R = 384
_S = 2048
_D = 128
_BH = 32
_RB = 8


def _row_body(q, kk):
    c_row = lax.dot_general(q, kk, (((1,), (1,)), ((), ())),
                            preferred_element_type=jnp.float32)
    c_col = lax.transpose(c_row[:, 1:_R + 1], (1, 0))

    base = jnp.sum((c_row < c_col).astype(jnp.int32),
                   axis=1, keepdims=True)
    cL = c_row[:, :_R]
    jT = lax.broadcasted_iota(jnp.int32, (_R, _R), 1)
    iT = lax.broadcasted_iota(jnp.int32, (_R, _R), 0) + 1
    tie = jnp.sum(((cL == c_col) & (jT < iT)).astype(jnp.int32),
                  axis=1, keepdims=True)
    c0 = c_row[:, :1]
    corr = (c0 <= c_col).astype(jnp.int32)
    pos = base + tie - corr + 1

    j2i = lax.broadcasted_iota(jnp.int32, (_R, _S), 1)
    hit = jnp.any(j2i == pos, axis=0, keepdims=True)
    row0 = lax.broadcasted_iota(jnp.int32, (1, _S), 1) == 0
    return jnp.where(hit | row0, 1, 0).astype(jnp.int32)


def _block_kernel(q_ref, k_ref, out_ref):
    for r in range(_RB):
        out_ref[r] = _row_body(q_ref[r], k_ref[r])


def kernel(q, k):
    q0 = q[:, :1, :]
    mask_i32 = pl.pallas_call(
        _block_kernel,
        grid=(_BH // _RB,),
        in_specs=[
            pl.BlockSpec((_RB, 1, _D), lambda b: (b, 0, 0)),
            pl.BlockSpec((_RB, _S, _D), lambda b: (b, 0, 0)),
        ],
        out_specs=pl.BlockSpec((_RB, 1, _S), lambda b: (b, 0, 0)),
        out_shape=jax.ShapeDtypeStruct((_BH, 1, _S), jnp.int32),
    )(q0, k)
    return mask_i32[:, 0, :] != 0
